# Initial kernel scaffold; baseline (speedup 1.0000x reference)
import jax
import jax.numpy as jnp
from jax.experimental import pallas as pl


def kernel(nucleotide_ids):
    raise NotImplementedError("write your pallas kernel here")


# speedup vs baseline: 408.2018x; 408.2018x over previous
import functools
import numpy as np
import jax
import jax.numpy as jnp
from jax import lax
from jax.experimental import pallas as pl
from jax.experimental.pallas import tpu as pltpu
from jax.experimental.pallas import tpu_sc as plsc

_PAD_ID = 22
_X_ID = 21
_STOP_ID = 20
_B = 4096
_L = 1536
_NWORKERS = 32
_ROWS_PER = _B // _NWORKERS


def _packed_table():
    code = "FFLLSSSSYY**CC*WLLLLPPPPHHQQRRRRIIIMTTTTNNKKSSRRVVVVAAAADDEEGGGG"
    aa_order = "ACDEFGHIKLMNPQRSTVWY"
    aa_to_id = {a: i for i, a in enumerate(aa_order)}
    aa_to_id["*"] = _STOP_ID
    tab = np.full((5, 5, 5), _X_ID, dtype=np.int32)
    idx_map = {0: 2, 1: 1, 2: 3, 3: 0}
    for a in range(4):
        for b in range(4):
            for c in range(4):
                k = idx_map[a] * 16 + idx_map[b] * 4 + idx_map[c]
                tab[a, b, c] = aa_to_id[code[k]]
    rc = np.array([3, 2, 1, 0, 4])
    packed = np.zeros(128, dtype=np.int32)
    for a in range(5):
        for b in range(5):
            for c in range(5):
                fwd = tab[a, b, c]
                rev = tab[rc[c], rc[b], rc[a]]
                packed[a * 25 + b * 5 + c] = fwd | (rev << 5)
    return packed


_CTAB = _packed_table()

_CLASS_PLAN = ((0, 3, 511), (2, 4, 510), (1, 5, 510))


def _sc_body(nuc_hbm, ctab_hbm, out_hbm, rowbuf, outbuf, ctab_v):
    wid = lax.axis_index("s") * 2 + lax.axis_index("c")
    pltpu.sync_copy(ctab_hbm, ctab_v)
    rowbuf[pl.ds(_L, 16)] = jnp.zeros((16,), jnp.int32)
    lane = lax.iota(jnp.int32, 16)
    iota3 = lane * 3
    pad = jnp.int32(_PAD_ID)
    base_row = wid * _ROWS_PER
    pad_slots = jnp.where(lane == 0, 4 * 512 + 511, 5 * 512 + 511)
    plsc.store_scatter(outbuf, [pad_slots], jnp.full((16,), pad), mask=lane < 2)

    def row_body(r, carry):
        row = base_row + r
        pltpu.sync_copy(nuc_hbm.at[row], rowbuf.at[pl.ds(0, _L)])

        def m_body(m, carry2):
            posv = m * 48 + iota3
            g0 = plsc.load_gather(rowbuf, [posv])
            g1 = plsc.load_gather(rowbuf, [posv + 1])
            g2 = plsc.load_gather(rowbuf, [posv + 2])
            g3 = plsc.load_gather(rowbuf, [posv + 3])
            g4 = plsc.load_gather(rowbuf, [posv + 4])
            gs = (g0, g1, g2, g3, g4)
            col = m * 16
            kv = col + lane
            for c, fr_rev, k0 in _CLASS_PLAN:
                idx = gs[c] * 25 + gs[c + 1] * 5 + gs[c + 2]
                ct = plsc.load_gather(ctab_v, [idx])
                fv = ct & 31
                rv = lax.shift_right_logical(ct, 5)
                if c != 0:
                    fv = jnp.where(kv == 511, pad, fv)
                outbuf[pl.ds(c * 512 + col, 16)] = fv
                ridx = (fr_rev * 512 + k0) - kv
                if c == 0:
                    plsc.store_scatter(outbuf, [ridx], rv)
                else:
                    plsc.store_scatter(outbuf, [ridx], rv, mask=kv <= k0)
            return carry2

        lax.fori_loop(0, 32, m_body, 0, unroll=2)
        pltpu.sync_copy(outbuf, out_hbm.at[row])
        return carry

    lax.fori_loop(0, _ROWS_PER, row_body, 0)


def kernel(nucleotide_ids):
    ctab = jnp.asarray(_CTAB)
    mesh = plsc.VectorSubcoreMesh(core_axis_name="c", subcore_axis_name="s")
    run = pl.kernel(
        _sc_body,
        out_type=jax.ShapeDtypeStruct((_B, 6 * 512), jnp.int32),
        mesh=mesh,
        scratch_types=[
            pltpu.VMEM((_L + 16,), jnp.int32),
            pltpu.VMEM((6 * 512,), jnp.int32),
            pltpu.VMEM((128,), jnp.int32),
        ],
        compiler_params=pltpu.CompilerParams(needs_layout_passes=False),
    )
    flat = run(nucleotide_ids, ctab)
    aa_ids = flat.reshape(_B, 6, 512)
    frame_lengths = jnp.broadcast_to(
        jnp.asarray([512, 511, 511, 512, 511, 511], jnp.int32), (_B, 6)
    )
    return aa_ids, frame_lengths


# speedup vs baseline: 606.8735x; 1.4867x over previous
import functools
import numpy as np
import jax
import jax.numpy as jnp
from jax import lax
from jax.experimental import pallas as pl
from jax.experimental.pallas import tpu as pltpu
from jax.experimental.pallas import tpu_sc as plsc

_PAD_ID = 22
_X_ID = 21
_STOP_ID = 20
_B = 4096
_L = 1536
_NWORKERS = 32
_ROWS_PER = _B // _NWORKERS


def _packed_table():
    code = "FFLLSSSSYY**CC*WLLLLPPPPHHQQRRRRIIIMTTTTNNKKSSRRVVVVAAAADDEEGGGG"
    aa_order = "ACDEFGHIKLMNPQRSTVWY"
    aa_to_id = {a: i for i, a in enumerate(aa_order)}
    aa_to_id["*"] = _STOP_ID
    tab = np.full((5, 5, 5), _X_ID, dtype=np.int32)
    idx_map = {0: 2, 1: 1, 2: 3, 3: 0}
    for a in range(4):
        for b in range(4):
            for c in range(4):
                k = idx_map[a] * 16 + idx_map[b] * 4 + idx_map[c]
                tab[a, b, c] = aa_to_id[code[k]]
    rc = np.array([3, 2, 1, 0, 4])
    packed = np.zeros(128, dtype=np.int32)
    for a in range(5):
        for b in range(5):
            for c in range(5):
                fwd = tab[a, b, c]
                rev = tab[rc[c], rc[b], rc[a]]
                packed[a * 25 + b * 5 + c] = fwd | (rev << 5)
    return packed


_CTAB = _packed_table()

_CLASS_PLAN = ((0, 3, 511), (2, 4, 510), (1, 5, 510))


def _sc_body(
    nuc_hbm, ctab_hbm, out_hbm, rb0, rb1, ob0, ob1, ctab_v, si0, si1, so0, so1
):
    wid = lax.axis_index("s") * 2 + lax.axis_index("c")
    pltpu.sync_copy(ctab_hbm, ctab_v)
    lane = lax.iota(jnp.int32, 16)
    iota3 = lane * 3
    pad = jnp.int32(_PAD_ID)
    base_row = wid * _ROWS_PER
    pad_slots = jnp.where(lane == 0, 4 * 512 + 511, 5 * 512 + 511)
    zeros16 = jnp.zeros((16,), jnp.int32)
    for rb, ob in ((rb0, ob0), (rb1, ob1)):
        rb[pl.ds(_L, 16)] = zeros16
        plsc.store_scatter(ob, [pad_slots], jnp.full((16,), pad), mask=lane < 2)

    def in_copy(row, rb, sem):
        return pltpu.make_async_copy(nuc_hbm.at[row], rb.at[pl.ds(0, _L)], sem)

    def out_copy(row, ob, sem):
        return pltpu.make_async_copy(ob, out_hbm.at[row], sem)

    def compute_row(rowbuf, outbuf):
        def m_body(m, carry2):
            posv = m * 48 + iota3
            g0 = plsc.load_gather(rowbuf, [posv])
            g1 = plsc.load_gather(rowbuf, [posv + 1])
            g2 = plsc.load_gather(rowbuf, [posv + 2])
            g3 = plsc.load_gather(rowbuf, [posv + 3])
            g4 = plsc.load_gather(rowbuf, [posv + 4])
            gs = (g0, g1, g2, g3, g4)
            col = m * 16
            kv = col + lane
            for c, fr_rev, k0 in _CLASS_PLAN:
                idx = gs[c] * 25 + gs[c + 1] * 5 + gs[c + 2]
                ct = plsc.load_gather(ctab_v, [idx])
                fv = ct & 31
                rv = lax.shift_right_logical(ct, 5)
                if c != 0:
                    fv = jnp.where(kv == 511, pad, fv)
                outbuf[pl.ds(c * 512 + col, 16)] = fv
                ridx = (fr_rev * 512 + k0) - kv
                if c == 0:
                    plsc.store_scatter(outbuf, [ridx], rv)
                else:
                    plsc.store_scatter(outbuf, [ridx], rv, mask=kv <= k0)
            return carry2

        lax.fori_loop(0, 32, m_body, 0, unroll=2)

    in_copy(base_row, rb0, si0).start()
    in_copy(base_row + 1, rb1, si1).start()

    def pair_body(i, carry):
        for s, (rb, ob, si, so) in enumerate(
            ((rb0, ob0, si0, so0), (rb1, ob1, si1, so1))
        ):
            row = base_row + 2 * i + s
            in_copy(row, rb, si).wait()

            @pl.when(i > 0)
            def _():
                out_copy(row, ob, so).wait()

            compute_row(rb, ob)
            nxt = jnp.minimum(row + 2, _B - 1)
            in_copy(nxt, rb, si).start()
            out_copy(row, ob, so).start()
        return carry

    lax.fori_loop(0, _ROWS_PER // 2, pair_body, 0)
    last = base_row + _ROWS_PER - 2
    out_copy(last, ob0, so0).wait()
    out_copy(last + 1, ob1, so1).wait()
    in_copy(last, rb0, si0).wait()
    in_copy(last + 1, rb1, si1).wait()


def kernel(nucleotide_ids):
    ctab = jnp.asarray(_CTAB)
    mesh = plsc.VectorSubcoreMesh(core_axis_name="c", subcore_axis_name="s")
    run = pl.kernel(
        _sc_body,
        out_type=jax.ShapeDtypeStruct((_B, 6 * 512), jnp.int32),
        mesh=mesh,
        scratch_types=[
            pltpu.VMEM((_L + 16,), jnp.int32),
            pltpu.VMEM((_L + 16,), jnp.int32),
            pltpu.VMEM((6 * 512,), jnp.int32),
            pltpu.VMEM((6 * 512,), jnp.int32),
            pltpu.VMEM((128,), jnp.int32),
            pltpu.SemaphoreType.DMA,
            pltpu.SemaphoreType.DMA,
            pltpu.SemaphoreType.DMA,
            pltpu.SemaphoreType.DMA,
        ],
        compiler_params=pltpu.CompilerParams(needs_layout_passes=False),
    )
    flat = run(nucleotide_ids, ctab)
    aa_ids = flat.reshape(_B, 6, 512)
    frame_lengths = jnp.broadcast_to(
        jnp.asarray([512, 511, 511, 512, 511, 511], jnp.int32), (_B, 6)
    )
    return aa_ids, frame_lengths


# speedup vs baseline: 660.6525x; 1.0886x over previous
import functools
import numpy as np
import jax
import jax.numpy as jnp
from jax import lax
from jax.experimental import pallas as pl
from jax.experimental.pallas import tpu as pltpu
from jax.experimental.pallas import tpu_sc as plsc

_PAD_ID = 22
_X_ID = 21
_STOP_ID = 20
_B = 4096
_L = 1536
_NWORKERS = 32
_ROWS_PER = _B // _NWORKERS


def _packed_table():
    code = "FFLLSSSSYY**CC*WLLLLPPPPHHQQRRRRIIIMTTTTNNKKSSRRVVVVAAAADDEEGGGG"
    aa_order = "ACDEFGHIKLMNPQRSTVWY"
    aa_to_id = {a: i for i, a in enumerate(aa_order)}
    aa_to_id["*"] = _STOP_ID
    tab = np.full((5, 5, 5), _X_ID, dtype=np.int32)
    idx_map = {0: 2, 1: 1, 2: 3, 3: 0}
    for a in range(4):
        for b in range(4):
            for c in range(4):
                k = idx_map[a] * 16 + idx_map[b] * 4 + idx_map[c]
                tab[a, b, c] = aa_to_id[code[k]]
    rc = np.array([3, 2, 1, 0, 4])
    n = np.arange(5)
    a, b, c, d, e = np.meshgrid(n, n, n, n, n, indexing="ij")
    f0 = tab[a, b, c]
    r0 = tab[rc[c], rc[b], rc[a]]
    f1 = tab[b, c, d]
    r1 = tab[rc[d], rc[c], rc[b]]
    f2 = tab[c, d, e]
    r2 = tab[rc[e], rc[d], rc[c]]
    packed = f0 | (r0 << 5) | (f1 << 10) | (r1 << 15) | (f2 << 20) | (r2 << 25)
    out = np.zeros(3136, dtype=np.int32)
    out[:3125] = packed.reshape(-1)
    return out


_CTAB = _packed_table()


def _sc_body(
    nuc_hbm, ctab_hbm, out_hbm, rb0, rb1, ob0, ob1, ctab_v, si0, si1, so0, so1
):
    wid = lax.axis_index("s") * 2 + lax.axis_index("c")
    pltpu.sync_copy(ctab_hbm, ctab_v)
    lane = lax.iota(jnp.int32, 16)
    iota3 = lane * 3
    pad = jnp.int32(_PAD_ID)
    base_row = wid * _ROWS_PER
    pad_slots = jnp.where(lane == 0, 4 * 512 + 511, 5 * 512 + 511)
    zeros16 = jnp.zeros((16,), jnp.int32)
    for rb, ob in ((rb0, ob0), (rb1, ob1)):
        rb[pl.ds(_L, 16)] = zeros16
        plsc.store_scatter(ob, [pad_slots], jnp.full((16,), pad), mask=lane < 2)

    def in_copy(row, rb, sem):
        return pltpu.make_async_copy(nuc_hbm.at[row], rb.at[pl.ds(0, _L)], sem)

    def out_copy(row, ob, sem):
        return pltpu.make_async_copy(ob, out_hbm.at[row], sem)

    def compute_row(rowbuf, outbuf):
        def m_body(m, carry2):
            posv = m * 48 + iota3
            g0 = plsc.load_gather(rowbuf, [posv])
            g1 = plsc.load_gather(rowbuf, [posv + 1])
            g2 = plsc.load_gather(rowbuf, [posv + 2])
            g3 = plsc.load_gather(rowbuf, [posv + 3])
            g4 = plsc.load_gather(rowbuf, [posv + 4])
            idx5 = (((g0 * 5 + g1) * 5 + g2) * 5 + g3) * 5 + g4
            t = plsc.load_gather(ctab_v, [idx5])
            col = m * 16
            kv = col + lane
            five_bits = jnp.int32(31)
            f0 = t & five_bits
            r0 = lax.shift_right_logical(t, 5) & five_bits
            f1 = lax.shift_right_logical(t, 10) & five_bits
            r1 = lax.shift_right_logical(t, 15) & five_bits
            f2 = lax.shift_right_logical(t, 20) & five_bits
            r2 = lax.shift_right_logical(t, 25)
            outbuf[pl.ds(col, 16)] = f0
            outbuf[pl.ds(512 + col, 16)] = jnp.where(kv == 511, pad, f1)
            outbuf[pl.ds(1024 + col, 16)] = jnp.where(kv == 511, pad, f2)
            plsc.store_scatter(outbuf, [(3 * 512 + 511) - kv], r0)
            plsc.store_scatter(outbuf, [(5 * 512 + 510) - kv], r1, mask=kv <= 510)
            plsc.store_scatter(outbuf, [(4 * 512 + 510) - kv], r2, mask=kv <= 510)
            return carry2

        lax.fori_loop(0, 32, m_body, 0, unroll=4)

    in_copy(base_row, rb0, si0).start()
    in_copy(base_row + 1, rb1, si1).start()

    def pair_body(i, carry):
        for s, (rb, ob, si, so) in enumerate(
            ((rb0, ob0, si0, so0), (rb1, ob1, si1, so1))
        ):
            row = base_row + 2 * i + s
            in_copy(row, rb, si).wait()

            @pl.when(i > 0)
            def _():
                out_copy(row, ob, so).wait()

            compute_row(rb, ob)
            nxt = jnp.minimum(row + 2, _B - 1)
            in_copy(nxt, rb, si).start()
            out_copy(row, ob, so).start()
        return carry

    lax.fori_loop(0, _ROWS_PER // 2, pair_body, 0)
    last = base_row + _ROWS_PER - 2
    out_copy(last, ob0, so0).wait()
    out_copy(last + 1, ob1, so1).wait()
    in_copy(last, rb0, si0).wait()
    in_copy(last + 1, rb1, si1).wait()


def kernel(nucleotide_ids):
    ctab = jnp.asarray(_CTAB)
    mesh = plsc.VectorSubcoreMesh(core_axis_name="c", subcore_axis_name="s")
    run = pl.kernel(
        _sc_body,
        out_type=jax.ShapeDtypeStruct((_B, 6 * 512), jnp.int32),
        mesh=mesh,
        scratch_types=[
            pltpu.VMEM((_L + 16,), jnp.int32),
            pltpu.VMEM((_L + 16,), jnp.int32),
            pltpu.VMEM((6 * 512,), jnp.int32),
            pltpu.VMEM((6 * 512,), jnp.int32),
            pltpu.VMEM((3136,), jnp.int32),
            pltpu.SemaphoreType.DMA,
            pltpu.SemaphoreType.DMA,
            pltpu.SemaphoreType.DMA,
            pltpu.SemaphoreType.DMA,
        ],
        compiler_params=pltpu.CompilerParams(needs_layout_passes=False),
    )
    flat = run(nucleotide_ids, ctab)
    aa_ids = flat.reshape(_B, 6, 512)
    frame_lengths = jnp.broadcast_to(
        jnp.asarray([512, 511, 511, 512, 511, 511], jnp.int32), (_B, 6)
    )
    return aa_ids, frame_lengths
